# baseline (speedup 1.0000x reference)
import functools

import jax
import jax.numpy as jnp
from jax.experimental import pallas as pl
from jax.experimental.pallas import tpu as pltpu

B = 4096
D = 1024
FF = 2048
E = 8
K = 2
OUT = 1024
TB = 256


def _dense_body(x_ref, m_ref, wg_ref, bg_ref, w1_ref, b1_ref, w2_ref, b2_ref,
                out_ref, wfull_ref):
    e = pl.program_id(1)

    @pl.when(e == 0)
    def _gate():
        logits = jnp.dot(x_ref[...], wg_ref[...],
                         preferred_element_type=jnp.float32) + bg_ref[...]
        iota = jax.lax.broadcasted_iota(jnp.int32, (TB, E), 1)
        m1 = jnp.max(logits, axis=1, keepdims=True)
        i1 = jnp.min(jnp.where(logits == m1, iota, E + 1), axis=1,
                     keepdims=True)
        sel1 = iota == i1
        neg = jnp.where(sel1, -jnp.inf, logits)
        m2 = jnp.max(neg, axis=1, keepdims=True)
        i2 = jnp.min(jnp.where(neg == m2, iota, E + 1), axis=1, keepdims=True)
        sel2 = iota == i2
        eb = jnp.exp(m2 - m1)
        p1 = 1.0 / (1.0 + eb)
        p2 = eb / (1.0 + eb)
        wfull_ref[...] = (jnp.where(sel1, p1, 0.0)
                          + jnp.where(sel2, p2, 0.0)).astype(jnp.float32)

    xm = x_ref[...] * m_ref[...]
    h = jnp.maximum(
        jnp.dot(xm, w1_ref[0], preferred_element_type=jnp.float32)
        + b1_ref[0], 0.0)
    o = jnp.dot(h, w2_ref[0], preferred_element_type=jnp.float32) + b2_ref[0]
    iota = jax.lax.broadcasted_iota(jnp.int32, (TB, E), 1)
    w_col = jnp.sum(jnp.where(iota == e, wfull_ref[...], 0.0), axis=1,
                    keepdims=True)
    contrib = o * w_col

    @pl.when(e == 0)
    def _init():
        out_ref[...] = contrib

    @pl.when(e != 0)
    def _acc():
        out_ref[...] += contrib


@jax.jit
def kernel(insample_y, insample_mask, Wg, bg, W1, b1, W2, b2):
    bg2 = bg.reshape(1, E)
    b1r = b1.reshape(E, 1, FF)
    b2r = b2.reshape(E, 1, OUT)
    grid = (B // TB, E)
    return pl.pallas_call(
        _dense_body,
        grid=grid,
        in_specs=[
            pl.BlockSpec((TB, D), lambda t, e: (t, 0)),
            pl.BlockSpec((TB, D), lambda t, e: (t, 0)),
            pl.BlockSpec((D, E), lambda t, e: (0, 0)),
            pl.BlockSpec((1, E), lambda t, e: (0, 0)),
            pl.BlockSpec((1, D, FF), lambda t, e: (e, 0, 0)),
            pl.BlockSpec((1, 1, FF), lambda t, e: (e, 0, 0)),
            pl.BlockSpec((1, FF, OUT), lambda t, e: (e, 0, 0)),
            pl.BlockSpec((1, 1, OUT), lambda t, e: (e, 0, 0)),
        ],
        out_specs=pl.BlockSpec((TB, OUT), lambda t, e: (t, 0)),
        out_shape=jax.ShapeDtypeStruct((B, OUT), jnp.float32),
        scratch_shapes=[pltpu.VMEM((TB, E), jnp.float32)],
        compiler_params=pltpu.CompilerParams(
            dimension_semantics=("parallel", "arbitrary"),
        ),
    )(insample_y, insample_mask, Wg, bg2, W1, b1r, W2, b2r)


# speedup vs baseline: 2.4040x; 2.4040x over previous
import functools

import jax
import jax.numpy as jnp
from jax import lax
from jax.experimental import pallas as pl
from jax.experimental.pallas import tpu as pltpu
from jax.experimental.pallas import tpu_sc as plsc

B = 4096
D = 1024
FF = 2048
E = 8
K = 2
OUT = 1024
TB = 256
BS = 256
NB = 40
NTOT = NB * BS
NW = 32
TPW = B // NW
NA = B * K

_SC_MESH = dict(core_axis_name="c", subcore_axis_name="s",
                num_cores=2, num_subcores=16)
_SC_PARAMS = pltpu.CompilerParams(needs_layout_passes=False)


def _gate_body(x_ref, m_ref, wg_ref, bg_ref, xm_ref, eidx_ref, gw_ref):
    logits = jnp.dot(x_ref[...], wg_ref[...],
                     preferred_element_type=jnp.float32) + bg_ref[...]
    iota = lax.broadcasted_iota(jnp.int32, (TB, E), 1)
    m1 = jnp.max(logits, axis=1, keepdims=True)
    i1 = jnp.min(jnp.where(logits == m1, iota, E + 1), axis=1, keepdims=True)
    sel1 = iota == i1
    neg = jnp.where(sel1, -jnp.inf, logits)
    m2 = jnp.max(neg, axis=1, keepdims=True)
    i2 = jnp.min(jnp.where(neg == m2, iota, E + 1), axis=1, keepdims=True)
    eb = jnp.exp(m2 - m1)
    p1 = 1.0 / (1.0 + eb)
    p2 = eb / (1.0 + eb)
    eidx_ref[...] = jnp.concatenate([i1, i2], axis=1)
    gw_ref[...] = jnp.concatenate([p1, p2], axis=1)
    xm_ref[...] = x_ref[...] * m_ref[...]


def _gate(x, mask, Wg, bg):
    return pl.pallas_call(
        _gate_body,
        grid=(B // TB,),
        in_specs=[
            pl.BlockSpec((TB, D), lambda t: (t, 0)),
            pl.BlockSpec((TB, D), lambda t: (t, 0)),
            pl.BlockSpec((D, E), lambda t: (0, 0)),
            pl.BlockSpec((1, E), lambda t: (0, 0)),
        ],
        out_specs=[
            pl.BlockSpec((TB, D), lambda t: (t, 0)),
            pl.BlockSpec((TB, K), lambda t: (t, 0)),
            pl.BlockSpec((TB, K), lambda t: (t, 0)),
        ],
        out_shape=[
            jax.ShapeDtypeStruct((B, D), jnp.float32),
            jax.ShapeDtypeStruct((B, K), jnp.int32),
            jax.ShapeDtypeStruct((B, K), jnp.float32),
        ],
        compiler_params=pltpu.CompilerParams(
            dimension_semantics=("parallel",),
        ),
    )(x, mask, Wg, bg.reshape(1, E))


def _hist_body(ef_hbm, cnt_hbm, e_v, cnt_v, sem):
    wid = lax.axis_index("s") * 2 + lax.axis_index("c")
    lane = lax.broadcasted_iota(jnp.int32, (16,), 0)
    pltpu.sync_copy(ef_hbm.at[pl.ds(wid * TPW * K, TPW * K)], e_v)
    cnt = jnp.zeros((16,), jnp.int32)
    for v in range(TPW * K // 16):
        ev = e_v[pl.ds(16 * v, 16)]
        for e in range(E):
            c = jnp.sum((ev == e).astype(jnp.int32))
            cnt = cnt + jnp.where(lane == e, c, 0)
    cnt_v[...] = cnt
    pltpu.sync_copy(cnt_v, cnt_hbm.at[pl.ds(wid * 16, 16)])


@functools.cache
def _hist_kernel():
    return pl.kernel(
        _hist_body,
        out_type=jax.ShapeDtypeStruct((NW * 16,), jnp.int32),
        mesh=plsc.VectorSubcoreMesh(**_SC_MESH),
        scratch_types=[
            pltpu.VMEM((TPW * K,), jnp.int32),
            pltpu.VMEM((16,), jnp.int32),
            pltpu.SemaphoreType.DMA,
        ],
        compiler_params=_SC_PARAMS,
    )


def _plan_body(cnt_ref, base_ref, blk_ref):
    cnt = cnt_ref[...]
    wi = lax.broadcasted_iota(jnp.int32, (NW, NW), 0)
    wj = lax.broadcasted_iota(jnp.int32, (NW, NW), 1)
    lt = (wj < wi).astype(jnp.float32)
    s = jnp.dot(lt, cnt.astype(jnp.float32),
                preferred_element_type=jnp.float32).astype(jnp.int32)
    ctot = jnp.sum(cnt, axis=0, keepdims=True)
    nblk = (ctot + (BS - 1)) // BS
    li = lax.broadcasted_iota(jnp.int32, (16, 16), 0)
    lj = lax.broadcasted_iota(jnp.int32, (16, 16), 1)
    le = (li <= lj).astype(jnp.float32)
    endb = jnp.dot(nblk.astype(jnp.float32), le,
                   preferred_element_type=jnp.float32).astype(jnp.int32)
    startb = endb - nblk
    base_ref[...] = startb * BS + s
    jv = lax.broadcasted_iota(jnp.int32, (1, 128), 1)
    acc = jnp.zeros((1, 128), jnp.int32)
    lane16 = lax.broadcasted_iota(jnp.int32, (1, 16), 1)
    for e in range(E):
        end_e = jnp.sum(jnp.where(lane16 == e, endb, 0))
        acc = acc + (jv >= end_e).astype(jnp.int32)
    nb_used = jnp.sum(jnp.where(lane16 == E - 1, endb, 0))
    blk_ref[...] = jnp.where(jv == 48, nb_used, jnp.minimum(acc, E - 1))


def _plan(cnt):
    return pl.pallas_call(
        _plan_body,
        grid=(1,),
        in_specs=[pl.BlockSpec((NW, 16), lambda i: (0, 0))],
        out_specs=[
            pl.BlockSpec((NW, 16), lambda i: (0, 0)),
            pl.BlockSpec((1, 128), lambda i: (0, 0)),
        ],
        out_shape=[
            jax.ShapeDtypeStruct((NW, 16), jnp.int32),
            jax.ShapeDtypeStruct((1, 128), jnp.int32),
        ],
    )(cnt.reshape(NW, 16))


def _scatter_body(xm_hbm, ef_hbm, basearr_hbm, xs_hbm, pos_hbm,
                  e_v, base_v, dpos_v, rows_v, sem):
    wid = lax.axis_index("s") * 2 + lax.axis_index("c")
    base = wid * TPW
    lane = lax.broadcasted_iota(jnp.int32, (16,), 0)

    pltpu.sync_copy(ef_hbm.at[pl.ds(base * K, TPW * K)], e_v)
    pltpu.sync_copy(basearr_hbm.at[pl.ds(wid * 16, 16)], base_v)
    base_row = base_v[...]

    run = [jnp.int32(0)] * E
    for v in range(TPW * K // 16):
        ev = e_v[pl.ds(16 * v, 16)]
        dest = jnp.zeros((16,), jnp.int32)
        for e in range(E):
            mi = (ev == e).astype(jnp.int32)
            pref = plsc.cumsum(mi) - mi
            base_e = jnp.sum(jnp.where(lane == e, base_row, 0))
            dest = dest + mi * (base_e + run[e] + pref)
            run[e] = run[e] + jnp.sum(mi)
        dpos_v[pl.ds(16 * v, 16)] = dest

    pltpu.sync_copy(dpos_v, pos_hbm.at[pl.ds(base * K, TPW * K)])

    for c in range(TPW // 16):
        pltpu.sync_copy(xm_hbm.at[pl.ds(base + 16 * c, 16)], rows_v)
        i0 = plsc.load_gather(dpos_v, [32 * c + 2 * lane])
        i1 = plsc.load_gather(dpos_v, [32 * c + 2 * lane + 1])
        cp0 = pltpu.async_copy(rows_v, xs_hbm.at[i0], sem)
        cp1 = pltpu.async_copy(rows_v, xs_hbm.at[i1], sem)
        cp0.wait()
        cp1.wait()


@functools.cache
def _scatter_kernel():
    return pl.kernel(
        _scatter_body,
        out_type=[
            jax.ShapeDtypeStruct((NTOT, D), jnp.float32),
            jax.ShapeDtypeStruct((NA,), jnp.int32),
        ],
        mesh=plsc.VectorSubcoreMesh(**_SC_MESH),
        scratch_types=[
            pltpu.VMEM((TPW * K,), jnp.int32),
            pltpu.VMEM((16,), jnp.int32),
            pltpu.VMEM((TPW * K,), jnp.int32),
            pltpu.VMEM((16, D), jnp.float32),
            pltpu.SemaphoreType.DMA,
        ],
        compiler_params=_SC_PARAMS,
    )


def _gmm_body(s_ref, xs_ref, w1_ref, b1_ref, w2_ref, b2_ref, ys_ref):
    j = pl.program_id(0)

    @pl.when(j < s_ref[48])
    def _():
        h = jnp.maximum(
            jnp.dot(xs_ref[...], w1_ref[0],
                    preferred_element_type=jnp.float32) + b1_ref[0], 0.0)
        ys_ref[...] = jnp.dot(h, w2_ref[0],
                              preferred_element_type=jnp.float32) + b2_ref[0]


def _gmm(blk, xs, W1, b1, W2, b2):
    grid_spec = pltpu.PrefetchScalarGridSpec(
        num_scalar_prefetch=1,
        grid=(NB,),
        in_specs=[
            pl.BlockSpec((BS, D), lambda j, s: (j, 0)),
            pl.BlockSpec((1, D, FF), lambda j, s: (s[j], 0, 0)),
            pl.BlockSpec((1, 1, FF), lambda j, s: (s[j], 0, 0)),
            pl.BlockSpec((1, FF, OUT), lambda j, s: (s[j], 0, 0)),
            pl.BlockSpec((1, 1, OUT), lambda j, s: (s[j], 0, 0)),
        ],
        out_specs=pl.BlockSpec((BS, OUT), lambda j, s: (j, 0)),
    )
    return pl.pallas_call(
        _gmm_body,
        grid_spec=grid_spec,
        out_shape=jax.ShapeDtypeStruct((NTOT, OUT), jnp.float32),
        compiler_params=pltpu.CompilerParams(
            dimension_semantics=("arbitrary",),
        ),
    )(blk, xs, W1, b1.reshape(E, 1, FF), W2, b2.reshape(E, 1, OUT))


def _combine_body(ys_hbm, pos_hbm, gw_hbm, out_hbm,
                  pos_v, w_v, r0_v, r1_v, o_v, sem):
    wid = lax.axis_index("s") * 2 + lax.axis_index("c")
    base = wid * TPW
    lane = lax.broadcasted_iota(jnp.int32, (16,), 0)

    pltpu.sync_copy(pos_hbm.at[pl.ds(base * K, TPW * K)], pos_v)
    pltpu.sync_copy(gw_hbm.at[pl.ds(base * K, TPW * K)], w_v)

    for c in range(TPW // 16):
        i0 = plsc.load_gather(pos_v, [32 * c + 2 * lane])
        i1 = plsc.load_gather(pos_v, [32 * c + 2 * lane + 1])
        cp0 = pltpu.async_copy(ys_hbm.at[i0], r0_v, sem)
        cp1 = pltpu.async_copy(ys_hbm.at[i1], r1_v, sem)
        cp0.wait()
        cp1.wait()

        def row_body(r, carry):
            w0 = plsc.load_gather(w_v, [jnp.zeros((16,), jnp.int32)
                                        + (32 * c + 2 * r)])
            w1 = plsc.load_gather(w_v, [jnp.zeros((16,), jnp.int32)
                                        + (32 * c + 2 * r + 1)])

            def col_body(k, carry2):
                a = r0_v[r, pl.ds(16 * k, 16)]
                b = r1_v[r, pl.ds(16 * k, 16)]
                o_v[r, pl.ds(16 * k, 16)] = w0 * a + w1 * b
                return carry2

            return lax.fori_loop(0, OUT // 16, col_body, carry)

        lax.fori_loop(0, 16, row_body, 0)
        pltpu.sync_copy(o_v, out_hbm.at[pl.ds(base + 16 * c, 16)])


@functools.cache
def _combine_kernel():
    return pl.kernel(
        _combine_body,
        out_type=jax.ShapeDtypeStruct((B, OUT), jnp.float32),
        mesh=plsc.VectorSubcoreMesh(**_SC_MESH),
        scratch_types=[
            pltpu.VMEM((TPW * K,), jnp.int32),
            pltpu.VMEM((TPW * K,), jnp.float32),
            pltpu.VMEM((16, OUT), jnp.float32),
            pltpu.VMEM((16, OUT), jnp.float32),
            pltpu.VMEM((16, OUT), jnp.float32),
            pltpu.SemaphoreType.DMA,
        ],
        compiler_params=_SC_PARAMS,
    )


@jax.jit
def kernel(insample_y, insample_mask, Wg, bg, W1, b1, W2, b2):
    xm, eidx, gw = _gate(insample_y, insample_mask, Wg, bg)
    ef = eidx.reshape(NA)
    cnt = _hist_kernel()(ef)
    basearr, blk = _plan(cnt)
    xs, pos = _scatter_kernel()(xm, ef, basearr.reshape(NW * 16))
    ys = _gmm(blk.reshape(128), xs, W1, b1, W2, b2)
    return _combine_kernel()(ys, pos, gw.reshape(NA))
